# Initial kernel scaffold; baseline (speedup 1.0000x reference)
import functools

import jax
import jax.numpy as jnp
from jax import lax
from jax.experimental import pallas as pl
from jax.experimental.pallas import tpu as pltpu
from jax.experimental.pallas import tpu_sc as plsc

_N = 10000
_E = 320000
_F = 128
_K = 64
_CUTOFF = 10.0
_LN2 = 0.6931471805599453

_NC = 2
_NS = 16
_NW = _NC * _NS
_EPW = _E // _NW
_NPS = _N // _NS


def _ssp(v):
    return jnp.maximum(v, 0.0) + jnp.log(1.0 + jnp.exp(-jnp.abs(v))) - _LN2


def _sc_gather_geom(R4, idx_i, idx_j):
    C = 80
    mesh = plsc.VectorSubcoreMesh(core_axis_name="c", subcore_axis_name="s")

    @functools.partial(
        pl.kernel,
        out_type=(jax.ShapeDtypeStruct((_E, 4), jnp.float32),
                  jax.ShapeDtypeStruct((_E, 4), jnp.float32)),
        mesh=mesh,
        scratch_types=[
            pltpu.VMEM((C,), jnp.int32),
            pltpu.VMEM((C, 4), jnp.float32),
            pltpu.SemaphoreType.DMA,
        ],
    )
    def k(r_hbm, ii_hbm, ij_hbm, ri_hbm, rj_hbm, idx_v, rows_v, sem):
        wid = lax.axis_index("c") * _NS + lax.axis_index("s")
        base0 = wid * _EPW

        @pl.loop(0, _EPW, step=C)
        def _(off):
            b = base0 + off
            for src, dst in ((ii_hbm, ri_hbm), (ij_hbm, rj_hbm)):
                pltpu.sync_copy(src.at[pl.ds(b, C)], idx_v)
                pltpu.async_copy(r_hbm.at[idx_v], rows_v, sem).wait()
                pltpu.sync_copy(rows_v, dst.at[pl.ds(b, C)])

    return k(R4, idx_i, idx_j)


def _sc_edge_block(G, Y, idx_i, idx_j, zeros_slab):
    C = 80
    mesh = plsc.VectorSubcoreMesh(core_axis_name="c", subcore_axis_name="s")

    @functools.partial(
        pl.kernel,
        out_type=jax.ShapeDtypeStruct((_NC, _N, _F), jnp.float32),
        mesh=mesh,
        scratch_types=[
            pltpu.VMEM((C,), jnp.int32),
            pltpu.VMEM((C,), jnp.int32),
            pltpu.VMEM((C, _F), jnp.float32),
            pltpu.VMEM((C, _F), jnp.float32),
            pltpu.VMEM_SHARED((_N, _F), jnp.float32),
            pltpu.SemaphoreType.DMA,
        ],
    )
    def k(g_hbm, y_hbm, ii_hbm, ij_hbm, z_hbm, out_hbm,
          ij_v, ii_v, y_v, g_v, acc, sem):
        c = lax.axis_index("c")
        s = lax.axis_index("s")
        wid = c * _NS + s

        pltpu.sync_copy(z_hbm, acc.at[pl.ds(s * _NPS, _NPS)])
        plsc.subcore_barrier()

        base0 = wid * _EPW

        @pl.loop(0, _EPW, step=C)
        def _(off):
            b = base0 + off
            pltpu.sync_copy(ij_hbm.at[pl.ds(b, C)], ij_v)
            pltpu.sync_copy(ii_hbm.at[pl.ds(b, C)], ii_v)
            pltpu.async_copy(y_hbm.at[ij_v], y_v, sem).wait()
            pltpu.sync_copy(g_hbm.at[pl.ds(b, C)], g_v)

            @pl.loop(0, C)
            def _(r):
                for q in range(_F // 16):
                    sl = (r, pl.ds(q * 16, 16))
                    y_v.at[sl][...] = y_v.at[sl][...] * g_v.at[sl][...]

            pltpu.sync_copy(y_v, acc.at[ii_v], add=True)

        plsc.subcore_barrier()
        pltpu.sync_copy(acc.at[pl.ds(s * _NPS, _NPS)],
                        out_hbm.at[c, pl.ds(s * _NPS, _NPS)])

    return k(G, Y, idx_i, idx_j, zeros_slab)


def _tc_embed(Z2, emb_pad):
    bN = 500

    def body(z_ref, emb_ref, o_ref):
        z = z_ref[...]
        kk = lax.broadcasted_iota(jnp.int32, (bN, 128), 1)
        oh = (kk == z).astype(jnp.float32)
        o_ref[...] = jnp.dot(oh, emb_ref[...],
                             preferred_element_type=jnp.float32)

    return pl.pallas_call(
        body,
        grid=(_N // bN,),
        in_specs=[pl.BlockSpec((bN, 1), lambda i: (i, 0)),
                  pl.BlockSpec((128, _F), lambda i: (0, 0))],
        out_specs=pl.BlockSpec((bN, _F), lambda i: (i, 0)),
        out_shape=jax.ShapeDtypeStruct((_N, _F), jnp.float32),
    )(Z2, emb_pad)


def _tc_gmat(Ri, Rj, k2f, centers, widths):
    bE = 1000

    def body(ri_ref, rj_ref, w_ref, c_ref, wd_ref, g_ref):
        d = ri_ref[...] - rj_ref[...]
        d2 = jnp.sum(d * d, axis=1, keepdims=True)
        D = jnp.sqrt(d2 + 1e-10)
        xc = D * (1.0 / _CUTOFF)
        x3 = xc * xc * xc
        x4 = x3 * xc
        x5 = x4 * xc
        cut = jnp.where(D < _CUTOFF,
                        1.0 - 6.0 * x5 + 15.0 * x4 - 10.0 * x3, 0.0)
        t = jnp.exp(-D)
        rbf = cut * jnp.exp(-wd_ref[...] * (t - c_ref[...]) ** 2)
        g_ref[...] = jnp.dot(rbf, w_ref[...],
                             preferred_element_type=jnp.float32)

    return pl.pallas_call(
        body,
        grid=(_E // bE,),
        in_specs=[pl.BlockSpec((bE, 4), lambda i: (i, 0)),
                  pl.BlockSpec((bE, 4), lambda i: (i, 0)),
                  pl.BlockSpec((_K, _F), lambda i: (0, 0)),
                  pl.BlockSpec((1, _K), lambda i: (0, 0)),
                  pl.BlockSpec((1, _K), lambda i: (0, 0))],
        out_specs=pl.BlockSpec((bE, _F), lambda i: (i, 0)),
        out_shape=jax.ShapeDtypeStruct((_E, _F), jnp.float32),
    )(Ri, Rj, k2f, centers, widths)


def _tc_pre(x, wj, bj, wi, bi):
    bN = 500

    def body(x_ref, wj_ref, bj_ref, wi_ref, bi_ref, y_ref, mi_ref):
        xa = _ssp(x_ref[...])
        y_ref[...] = jnp.dot(xa, wj_ref[...],
                             preferred_element_type=jnp.float32) + bj_ref[...]
        mi_ref[...] = jnp.dot(xa, wi_ref[...],
                              preferred_element_type=jnp.float32) + bi_ref[...]

    return pl.pallas_call(
        body,
        grid=(_N // bN,),
        in_specs=[pl.BlockSpec((bN, _F), lambda i: (i, 0)),
                  pl.BlockSpec((_F, _F), lambda i: (0, 0)),
                  pl.BlockSpec((1, _F), lambda i: (0, 0)),
                  pl.BlockSpec((_F, _F), lambda i: (0, 0)),
                  pl.BlockSpec((1, _F), lambda i: (0, 0))],
        out_specs=[pl.BlockSpec((bN, _F), lambda i: (i, 0)),
                   pl.BlockSpec((bN, _F), lambda i: (i, 0))],
        out_shape=[jax.ShapeDtypeStruct((_N, _F), jnp.float32),
                   jax.ShapeDtypeStruct((_N, _F), jnp.float32)],
    )(x, wj, bj, wi, bi)


def _tc_post(x, mi, xj2, ws, bs, u, wo, bo):
    bN = 500

    def body(x_ref, mi_ref, xj_ref, ws_ref, bs_ref, u_ref, wo_ref, bo_ref,
             xout_ref, out_ref):
        def dense(k, h):
            return jnp.dot(h, ws_ref[k],
                           preferred_element_type=jnp.float32) + bs_ref[k]

        def res(k, h):
            return h + dense(k + 1, _ssp(dense(k, _ssp(h))))

        m = mi_ref[...] + xj_ref[0] + xj_ref[1]
        m = res(0, m)
        m = res(2, m)
        m = _ssp(m)
        xn = u_ref[...] * x_ref[...] + dense(4, m)
        xn = res(5, xn)
        xn = res(7, xn)
        xout_ref[...] = xn
        h = res(9, xn)
        out_ref[...] = jnp.dot(_ssp(h), wo_ref[...],
                               preferred_element_type=jnp.float32) + bo_ref[...]

    return pl.pallas_call(
        body,
        grid=(_N // bN,),
        in_specs=[pl.BlockSpec((bN, _F), lambda i: (i, 0)),
                  pl.BlockSpec((bN, _F), lambda i: (i, 0)),
                  pl.BlockSpec((_NC, bN, _F), lambda i: (0, i, 0)),
                  pl.BlockSpec((11, _F, _F), lambda i: (0, 0, 0)),
                  pl.BlockSpec((11, _F), lambda i: (0, 0)),
                  pl.BlockSpec((1, _F), lambda i: (0, 0)),
                  pl.BlockSpec((_F, 2), lambda i: (0, 0)),
                  pl.BlockSpec((1, 2), lambda i: (0, 0))],
        out_specs=[pl.BlockSpec((bN, _F), lambda i: (i, 0)),
                   pl.BlockSpec((bN, 2), lambda i: (i, 0))],
        out_shape=[jax.ShapeDtypeStruct((_N, _F), jnp.float32),
                   jax.ShapeDtypeStruct((_N, 2), jnp.float32)],
    )(x, mi, xj2, ws, bs, u, wo, bo)


def _tc_final(o0, o1, o2):
    def body(r0, r1, r2, sum_ref, nh_ref):
        a, b, c = r0[...], r1[...], r2[...]
        sum_ref[...] = a + b + c
        a2, b2, c2 = a * a, b * b, c * c
        n1 = jnp.sum(b2 / (b2 + a2 + 1e-7))
        n2 = jnp.sum(c2 / (c2 + b2 + 1e-7))
        nh_ref[...] = ((n1 + n2) * (1.0 / (2.0 * _N))) * jnp.ones(
            (1, 1), jnp.float32)

    return pl.pallas_call(
        body,
        grid=(1,),
        in_specs=[pl.BlockSpec((_N, 2), lambda i: (0, 0))] * 3,
        out_specs=[pl.BlockSpec((_N, 2), lambda i: (0, 0)),
                   pl.BlockSpec((1, 1), lambda i: (0, 0))],
        out_shape=[jax.ShapeDtypeStruct((_N, 2), jnp.float32),
                   jax.ShapeDtypeStruct((1, 1), jnp.float32)],
    )(o0, o1, o2)


def kernel(R, Z, idx_i, idx_j, params):
    idx_i = idx_i.astype(jnp.int32)
    idx_j = idx_j.astype(jnp.int32)
    R4 = jnp.pad(R, ((0, 0), (0, 1)))
    Ri, Rj = _sc_gather_geom(R4, idx_i, idx_j)

    emb_pad = jnp.zeros((128, _F), jnp.float32).at[:95].set(
        params["embeddings"])
    x = _tc_embed(Z.reshape(_N, 1).astype(jnp.int32), emb_pad)

    centers = params["rbf_centers"].reshape(1, _K)
    widths = params["rbf_widths"].reshape(1, _K)
    zeros_slab = jnp.zeros((_NPS, _F), jnp.float32)

    outs = []
    for b in range(3):
        p = params["blocks"][b]
        G = _tc_gmat(Ri, Rj, p["k2f"]["W"], centers, widths)
        Y, Mi = _tc_pre(x, p["dense_j"]["W"], p["dense_j"]["b"].reshape(1, _F),
                        p["dense_i"]["W"], p["dense_i"]["b"].reshape(1, _F))
        XJ2 = _sc_edge_block(G, Y, idx_i, idx_j, zeros_slab)

        ri = p["res_interaction"]
        ra = p["res_atomic"]
        ro = p["res_output"]
        wlist = [ri[0]["d1"]["W"], ri[0]["d2"]["W"],
                 ri[1]["d1"]["W"], ri[1]["d2"]["W"],
                 p["dense_out"]["W"],
                 ra[0]["d1"]["W"], ra[0]["d2"]["W"],
                 ra[1]["d1"]["W"], ra[1]["d2"]["W"],
                 ro[0]["d1"]["W"], ro[0]["d2"]["W"]]
        blist = [ri[0]["d1"]["b"], ri[0]["d2"]["b"],
                 ri[1]["d1"]["b"], ri[1]["d2"]["b"],
                 p["dense_out"]["b"],
                 ra[0]["d1"]["b"], ra[0]["d2"]["b"],
                 ra[1]["d1"]["b"], ra[1]["d2"]["b"],
                 ro[0]["d1"]["b"], ro[0]["d2"]["b"]]
        ws = jnp.stack(wlist)
        bs = jnp.stack(blist)
        x, out_b = _tc_post(x, Mi, XJ2, ws, bs, p["u"].reshape(1, _F),
                            p["out_dense"]["W"],
                            p["out_dense"]["b"].reshape(1, 2))
        outs.append(out_b)

    sums, nh = _tc_final(*outs)
    return sums[:, 0], sums[:, 1], nh[0, 0]


# speedup vs baseline: 2.7679x; 2.7679x over previous
import dataclasses
import functools

import jax
import jax.numpy as jnp
from jax import lax
from jax.experimental import pallas as pl
from jax.experimental.pallas import tpu as pltpu
from jax.experimental.pallas import tpu_sc as plsc

_N = 10000
_E = 320000
_F = 128
_K = 64
_CUTOFF = 10.0
_LN2 = 0.6931471805599453

_NC = 2
_NS = 16
_NW = _NC * _NS
_EPW = _E // _NW
_NPAD = 10240
_NPS = _NPAD // _NS


def _sc_compiler_params():
    cp = pltpu.CompilerParams()
    if "needs_layout_passes" in pltpu.CompilerParams.__dataclass_fields__:
        cp = dataclasses.replace(cp, needs_layout_passes=False)
    return cp


def _ssp(v):
    return jnp.maximum(v, 0.0) + jnp.log(1.0 + jnp.exp(-jnp.abs(v))) - _LN2


def _sc_dist2(Rflat, idx_i, idx_j):
    mesh = plsc.VectorSubcoreMesh(core_axis_name="c", subcore_axis_name="s")

    @functools.partial(
        pl.kernel,
        out_type=jax.ShapeDtypeStruct((_E,), jnp.float32),
        mesh=mesh,
        scratch_types=[
            pltpu.VMEM((4 * _N,), jnp.float32),
            pltpu.VMEM((_EPW,), jnp.int32),
            pltpu.VMEM((_EPW,), jnp.int32),
            pltpu.VMEM((_EPW,), jnp.float32),
        ],
        compiler_params=_sc_compiler_params(),
    )
    def k(r_hbm, ii_hbm, ij_hbm, d2_hbm, r_v, ii_v, ij_v, d2_v):
        wid = lax.axis_index("c") * _NS + lax.axis_index("s")
        base = wid * _EPW
        pltpu.sync_copy(r_hbm, r_v)
        pltpu.sync_copy(ii_hbm.at[pl.ds(base, _EPW)], ii_v)
        pltpu.sync_copy(ij_hbm.at[pl.ds(base, _EPW)], ij_v)

        @pl.loop(0, _EPW, step=16)
        def _(r):
            ii16 = ii_v[pl.ds(r, 16)] * 4
            ij16 = ij_v[pl.ds(r, 16)] * 4
            acc = None
            for comp in range(3):
                a = plsc.load_gather(r_v, [ii16 + comp])
                b2 = plsc.load_gather(r_v, [ij16 + comp])
                d = a - b2
                acc = d * d if acc is None else acc + d * d
            d2_v[pl.ds(r, 16)] = acc

        pltpu.sync_copy(d2_v, d2_hbm.at[pl.ds(base, _EPW)])

    return k(Rflat, idx_i, idx_j)


_NB = 128


def _sc_edge_block(G, Y, idx_i, idx_j):
    C = 80
    mesh = plsc.VectorSubcoreMesh(core_axis_name="c", subcore_axis_name="s")

    @functools.partial(
        pl.kernel,
        out_type=jax.ShapeDtypeStruct((_NC, _NPAD, _F), jnp.float32),
        mesh=mesh,
        scratch_types=[
            pltpu.VMEM((C,), jnp.int32),
            pltpu.VMEM((C,), jnp.int32),
            pltpu.VMEM((C, _F), jnp.float32),
            pltpu.VMEM((C, _F), jnp.float32),
            pltpu.VMEM((_NB, _F), jnp.float32),
            pltpu.VMEM_SHARED((_NPAD, _F), jnp.float32),
            pltpu.SemaphoreType.DMA,
        ],
    )
    def k(g_hbm, y_hbm, ii_hbm, ij_hbm, out_hbm,
          ij_v, ii_v, y_v, g_v, buf_v, acc, sem):
        c = lax.axis_index("c")
        s = lax.axis_index("s")
        wid = c * _NS + s

        zero = jnp.zeros((16,), jnp.float32)

        @pl.loop(0, _NB)
        def _(r):
            for q in range(_F // 16):
                buf_v.at[r, pl.ds(q * 16, 16)][...] = zero

        for t in range(_NPS // _NB):
            pltpu.sync_copy(buf_v, acc.at[pl.ds(s * _NPS + t * _NB, _NB)])
        plsc.subcore_barrier()

        base0 = wid * _EPW

        @pl.loop(0, _EPW, step=C)
        def _(off):
            b = base0 + off
            pltpu.sync_copy(ij_hbm.at[pl.ds(b, C)], ij_v)
            pltpu.sync_copy(ii_hbm.at[pl.ds(b, C)], ii_v)
            pltpu.async_copy(y_hbm.at[ij_v], y_v, sem).wait()
            pltpu.sync_copy(g_hbm.at[pl.ds(b, C)], g_v)

            @pl.loop(0, C)
            def _(r):
                for q in range(_F // 16):
                    sl = (r, pl.ds(q * 16, 16))
                    y_v.at[sl][...] = y_v.at[sl][...] * g_v.at[sl][...]

            pltpu.sync_copy(y_v, acc.at[ii_v], add=True)

        plsc.subcore_barrier()
        for t in range(_NPS // _NB):
            pltpu.sync_copy(acc.at[pl.ds(s * _NPS + t * _NB, _NB)], buf_v)
            pltpu.sync_copy(buf_v,
                            out_hbm.at[c, pl.ds(s * _NPS + t * _NB, _NB)])

    return k(G, Y, idx_i, idx_j)


def _tc_embed(Z2, emb_pad):
    bN = 400

    def body(z_ref, emb_ref, o_ref):
        z = z_ref[...]
        kk = lax.broadcasted_iota(jnp.int32, (bN, 128), 1)
        oh = (kk == z).astype(jnp.float32)
        o_ref[...] = jnp.dot(oh, emb_ref[...],
                             preferred_element_type=jnp.float32)

    return pl.pallas_call(
        body,
        grid=(_N // bN,),
        in_specs=[pl.BlockSpec((bN, 1), lambda i: (i, 0)),
                  pl.BlockSpec((128, _F), lambda i: (0, 0))],
        out_specs=pl.BlockSpec((bN, _F), lambda i: (i, 0)),
        out_shape=jax.ShapeDtypeStruct((_N, _F), jnp.float32),
    )(Z2, emb_pad)


def _tc_gmat(d2col, k2f, centers, widths):
    bE = 1000

    def body(d2_ref, w_ref, c_ref, wd_ref, g_ref):
        D = jnp.sqrt(d2_ref[...] + 1e-10)
        xc = D * (1.0 / _CUTOFF)
        x3 = xc * xc * xc
        x4 = x3 * xc
        x5 = x4 * xc
        cut = jnp.where(D < _CUTOFF,
                        1.0 - 6.0 * x5 + 15.0 * x4 - 10.0 * x3, 0.0)
        t = jnp.exp(-D)
        rbf = cut * jnp.exp(-wd_ref[...] * (t - c_ref[...]) ** 2)
        g_ref[...] = jnp.dot(rbf, w_ref[...],
                             preferred_element_type=jnp.float32)

    return pl.pallas_call(
        body,
        grid=(_E // bE,),
        in_specs=[pl.BlockSpec((bE, 1), lambda i: (i, 0)),
                  pl.BlockSpec((_K, _F), lambda i: (0, 0)),
                  pl.BlockSpec((1, _K), lambda i: (0, 0)),
                  pl.BlockSpec((1, _K), lambda i: (0, 0))],
        out_specs=pl.BlockSpec((bE, _F), lambda i: (i, 0)),
        out_shape=jax.ShapeDtypeStruct((_E, _F), jnp.float32),
    )(d2col, k2f, centers, widths)


def _tc_pre(x, wj, bj, wi, bi):
    bN = 400

    def body(x_ref, wj_ref, bj_ref, wi_ref, bi_ref, y_ref, mi_ref):
        xa = _ssp(x_ref[...])
        y_ref[...] = jnp.dot(xa, wj_ref[...],
                             preferred_element_type=jnp.float32) + bj_ref[...]
        mi_ref[...] = jnp.dot(xa, wi_ref[...],
                              preferred_element_type=jnp.float32) + bi_ref[...]

    return pl.pallas_call(
        body,
        grid=(_N // bN,),
        in_specs=[pl.BlockSpec((bN, _F), lambda i: (i, 0)),
                  pl.BlockSpec((_F, _F), lambda i: (0, 0)),
                  pl.BlockSpec((1, _F), lambda i: (0, 0)),
                  pl.BlockSpec((_F, _F), lambda i: (0, 0)),
                  pl.BlockSpec((1, _F), lambda i: (0, 0))],
        out_specs=[pl.BlockSpec((bN, _F), lambda i: (i, 0)),
                   pl.BlockSpec((bN, _F), lambda i: (i, 0))],
        out_shape=[jax.ShapeDtypeStruct((_N, _F), jnp.float32),
                   jax.ShapeDtypeStruct((_N, _F), jnp.float32)],
    )(x, wj, bj, wi, bi)


def _tc_post(x, mi, xj2, ws, bs, u, wo, bo):
    bN = 400

    def body(x_ref, mi_ref, xj_ref, ws_ref, bs_ref, u_ref, wo_ref, bo_ref,
             xout_ref, out_ref):
        def dense(k, h):
            return jnp.dot(h, ws_ref[k],
                           preferred_element_type=jnp.float32) + bs_ref[k]

        def res(k, h):
            return h + dense(k + 1, _ssp(dense(k, _ssp(h))))

        m = mi_ref[...] + xj_ref[0] + xj_ref[1]
        m = res(0, m)
        m = res(2, m)
        m = _ssp(m)
        xn = u_ref[...] * x_ref[...] + dense(4, m)
        xn = res(5, xn)
        xn = res(7, xn)
        xout_ref[...] = xn
        h = res(9, xn)
        out_ref[...] = jnp.dot(_ssp(h), wo_ref[...],
                               preferred_element_type=jnp.float32) + bo_ref[...]

    return pl.pallas_call(
        body,
        grid=(_N // bN,),
        in_specs=[pl.BlockSpec((bN, _F), lambda i: (i, 0)),
                  pl.BlockSpec((bN, _F), lambda i: (i, 0)),
                  pl.BlockSpec((_NC, bN, _F), lambda i: (0, i, 0)),
                  pl.BlockSpec((11, _F, _F), lambda i: (0, 0, 0)),
                  pl.BlockSpec((11, _F), lambda i: (0, 0)),
                  pl.BlockSpec((1, _F), lambda i: (0, 0)),
                  pl.BlockSpec((_F, 2), lambda i: (0, 0)),
                  pl.BlockSpec((1, 2), lambda i: (0, 0))],
        out_specs=[pl.BlockSpec((bN, _F), lambda i: (i, 0)),
                   pl.BlockSpec((bN, 2), lambda i: (i, 0))],
        out_shape=[jax.ShapeDtypeStruct((_N, _F), jnp.float32),
                   jax.ShapeDtypeStruct((_N, 2), jnp.float32)],
    )(x, mi, xj2, ws, bs, u, wo, bo)


def _tc_final(o0, o1, o2):
    def body(r0, r1, r2, sum_ref, nh_ref):
        a, b, c = r0[...], r1[...], r2[...]
        sum_ref[...] = a + b + c
        a2, b2, c2 = a * a, b * b, c * c
        n1 = jnp.sum(b2 / (b2 + a2 + 1e-7))
        n2 = jnp.sum(c2 / (c2 + b2 + 1e-7))
        nh_ref[...] = ((n1 + n2) * (1.0 / (2.0 * _N))) * jnp.ones(
            (1, 1), jnp.float32)

    return pl.pallas_call(
        body,
        grid=(1,),
        in_specs=[pl.BlockSpec((_N, 2), lambda i: (0, 0))] * 3,
        out_specs=[pl.BlockSpec((_N, 2), lambda i: (0, 0)),
                   pl.BlockSpec((1, 1), lambda i: (0, 0))],
        out_shape=[jax.ShapeDtypeStruct((_N, 2), jnp.float32),
                   jax.ShapeDtypeStruct((1, 1), jnp.float32)],
    )(o0, o1, o2)


def kernel(R, Z, idx_i, idx_j, params):
    idx_i = idx_i.astype(jnp.int32)
    idx_j = idx_j.astype(jnp.int32)
    rflat = jnp.pad(R, ((0, 0), (0, 1))).reshape(4 * _N)
    d2col = _sc_dist2(rflat, idx_i, idx_j).reshape(_E, 1)

    emb_pad = jnp.zeros((128, _F), jnp.float32).at[:95].set(
        params["embeddings"])
    x = _tc_embed(Z.reshape(_N, 1).astype(jnp.int32), emb_pad)

    centers = params["rbf_centers"].reshape(1, _K)
    widths = params["rbf_widths"].reshape(1, _K)

    outs = []
    for b in range(3):
        p = params["blocks"][b]
        G = _tc_gmat(d2col, p["k2f"]["W"], centers, widths)
        Y, Mi = _tc_pre(x, p["dense_j"]["W"], p["dense_j"]["b"].reshape(1, _F),
                        p["dense_i"]["W"], p["dense_i"]["b"].reshape(1, _F))
        XJ2 = _sc_edge_block(G, Y, idx_i, idx_j)

        ri = p["res_interaction"]
        ra = p["res_atomic"]
        ro = p["res_output"]
        wlist = [ri[0]["d1"]["W"], ri[0]["d2"]["W"],
                 ri[1]["d1"]["W"], ri[1]["d2"]["W"],
                 p["dense_out"]["W"],
                 ra[0]["d1"]["W"], ra[0]["d2"]["W"],
                 ra[1]["d1"]["W"], ra[1]["d2"]["W"],
                 ro[0]["d1"]["W"], ro[0]["d2"]["W"]]
        blist = [ri[0]["d1"]["b"], ri[0]["d2"]["b"],
                 ri[1]["d1"]["b"], ri[1]["d2"]["b"],
                 p["dense_out"]["b"],
                 ra[0]["d1"]["b"], ra[0]["d2"]["b"],
                 ra[1]["d1"]["b"], ra[1]["d2"]["b"],
                 ro[0]["d1"]["b"], ro[0]["d2"]["b"]]
        ws = jnp.stack(wlist)
        bs = jnp.stack(blist)
        x, out_b = _tc_post(x, Mi, XJ2, ws, bs, p["u"].reshape(1, _F),
                            p["out_dense"]["W"],
                            p["out_dense"]["b"].reshape(1, 2))
        outs.append(out_b)

    sums, nh = _tc_final(*outs)
    return sums[:, 0], sums[:, 1], nh[0, 0]
